# baseline (speedup 1.0000x reference)
import functools

import jax
import jax.numpy as jnp
from jax import lax
from jax.experimental import pallas as pl
from jax.experimental.pallas import tpu as pltpu
from jax.experimental.pallas import tpu_sc as plsc

NC = 2
NS = 16
L = 16
NW = NC * NS

N = 100000
D = 128
ROWS_PER_W = N // NW
CHUNK = 125
NCHUNKS = ROWS_PER_W // CHUNK
NBUF = 4
GROUPS = (CHUNK + L - 1) // L
NFC = D // L

_NEG_INF = float("-inf")
_S_MIN = 1e-30


def _tree_sum(vs):
    while len(vs) > 1:
        vs = [a + b for a, b in zip(vs[::2], vs[1::2])]
    return vs[0]


def _sc_body(keys_hbm, q_hbm, values_hbm,
             rows_out, num_out, s_out, idx_out,
             q_vmem, buf, cand_n, cand_s, cand_i,
             merged_n, merged_s, merged_i,
             stat_vec, row_vmem,
             shared_n, shared_s, shared_i, sems):
    cid = lax.axis_index("c")
    sid = lax.axis_index("s")
    w = cid * NS + sid
    base = w * ROWS_PER_W

    pltpu.sync_copy(q_hbm, q_vmem)
    q_regs = [q_vmem[pl.ds(fc * L, L)] for fc in range(NFC)]

    def chunk_src(c):
        return keys_hbm.at[pl.ds(base + c * CHUNK, CHUNK)]

    def process(c, slot, carry):
        chunk_base = base + c * CHUNK

        def group(g, carry):
            bn, bs, bi = carry
            gbase = g * L
            for r in range(L):
                row = jnp.minimum(gbase + r, CHUNK - 1)
                kvs = [buf[slot, row, pl.ds(fc * L, L)] for fc in range(NFC)]
                d = jnp.sum(_tree_sum([kv * qv for kv, qv in zip(kvs, q_regs)]))
                s = jnp.sum(_tree_sum([kv * kv for kv in kvs]))
                s = jnp.maximum(s, _S_MIN)
                n = d * jnp.abs(d)
                upd = (n * bs > bn * s) & (gbase + r < CHUNK)
                bn = jnp.where(upd, n, bn)
                bs = jnp.where(upd, s, bs)
                bi = jnp.where(upd, chunk_base + gbase + r, bi)
            return bn, bs, bi

        return lax.fori_loop(0, GROUPS, group, carry)

    carry = (jnp.float32(_NEG_INF), jnp.float32(1.0), jnp.int32(0))

    for k in range(NBUF - 1):
        pltpu.async_copy(chunk_src(k), buf.at[k], sems.at[k])

    def step(c, carry):
        slot = lax.rem(c, NBUF)
        for k in range(NBUF):
            @pl.when(slot == k)
            def _():
                nc = c + NBUF - 1
                ns = (k + NBUF - 1) % NBUF

                @pl.when(nc < NCHUNKS)
                def _():
                    pltpu.async_copy(chunk_src(nc), buf.at[ns], sems.at[ns])

                pltpu.make_async_copy(chunk_src(c), buf.at[k], sems.at[k]).wait()

        return process(c, slot, carry)

    bn, bs, bi = lax.fori_loop(0, NCHUNKS, step, carry)

    cand_n[...] = jnp.full((L,), bn, jnp.float32)
    cand_s[...] = jnp.full((L,), bs, jnp.float32)
    cand_i[...] = jnp.full((L,), bi, jnp.int32)
    pltpu.sync_copy(cand_n, shared_n.at[sid])
    pltpu.sync_copy(cand_s, shared_s.at[sid])
    pltpu.sync_copy(cand_i, shared_i.at[sid])
    plsc.subcore_barrier()

    @pl.when(sid == 0)
    def _():
        pltpu.sync_copy(shared_n, merged_n)
        pltpu.sync_copy(shared_s, merged_s)
        pltpu.sync_copy(shared_i, merged_i)
        bn = merged_n[0, :]
        bs = merged_s[0, :]
        bi = merged_i[0, :]
        for t in range(1, NS):
            n = merged_n[t, :]
            s = merged_s[t, :]
            i = merged_i[t, :]
            a = n * bs
            b = bn * s
            upd = (a > b) | ((a == b) & (i < bi))
            bn = jnp.where(upd, n, bn)
            bs = jnp.where(upd, s, bs)
            bi = jnp.where(upd, i, bi)
        midx = jnp.max(bi)
        pltpu.sync_copy(values_hbm.at[pl.ds(midx, 1)], row_vmem)
        pltpu.sync_copy(row_vmem, rows_out.at[pl.ds(cid, 1)])
        stat_vec[...] = bn
        pltpu.sync_copy(stat_vec, num_out.at[cid])
        stat_vec[...] = bs
        pltpu.sync_copy(stat_vec, s_out.at[cid])
        cand_i[...] = bi
        pltpu.sync_copy(cand_i, idx_out.at[cid])


@jax.jit
def kernel(query, keys, values):
    mesh = plsc.VectorSubcoreMesh(core_axis_name="c", subcore_axis_name="s")
    rows, nums, ss, idxs = pl.kernel(
        _sc_body,
        out_type=(
            jax.ShapeDtypeStruct((NC, D), jnp.float32),
            jax.ShapeDtypeStruct((NC, L), jnp.float32),
            jax.ShapeDtypeStruct((NC, L), jnp.float32),
            jax.ShapeDtypeStruct((NC, L), jnp.int32),
        ),
        mesh=mesh,
        compiler_params=pltpu.CompilerParams(
            use_tc_tiling_on_sc=False, needs_layout_passes=False),
        scratch_types=[
            pltpu.VMEM((D,), jnp.float32),
            pltpu.VMEM((NBUF, CHUNK, D), jnp.float32),
            pltpu.VMEM((L,), jnp.float32),
            pltpu.VMEM((L,), jnp.float32),
            pltpu.VMEM((L,), jnp.int32),
            pltpu.VMEM((NS, L), jnp.float32),
            pltpu.VMEM((NS, L), jnp.float32),
            pltpu.VMEM((NS, L), jnp.int32),
            pltpu.VMEM((L,), jnp.float32),
            pltpu.VMEM((1, D), jnp.float32),
            pltpu.VMEM_SHARED((NS, L), jnp.float32),
            pltpu.VMEM_SHARED((NS, L), jnp.float32),
            pltpu.VMEM_SHARED((NS, L), jnp.int32),
            pltpu.SemaphoreType.DMA((NBUF,)),
        ],
    )(keys, query, values)

    n0, n1 = nums[0, 0], nums[1, 0]
    s0, s1 = ss[0, 0], ss[1, 0]
    i0, i1 = idxs[0, 0], idxs[1, 0]
    a, b = n0 * s1, n1 * s0
    pick0 = (a > b) | ((a == b) & (i0 <= i1))
    return jnp.where(pick0, rows[0], rows[1])
